# baseline (speedup 1.0000x reference)
import jax
import jax.numpy as jnp
from jax import lax
from jax.experimental import pallas as pl
from jax.experimental.pallas import tpu as pltpu
from jax.experimental.pallas import tpu_sc as plsc

NUM_CORES = 2
NUM_SUBCORES = 16
NUM_WORKERS = NUM_CORES * NUM_SUBCORES
BATCH = 16384
EMBED_DIM = 32
B_PER_W = BATCH // NUM_WORKERS
LANES = 16
CHUNK = 128
NCHUNKS = B_PER_W // CHUNK
CGROUPS = CHUNK // LANES


def _bpr_body(uidx_hbm, iidx_hbm, utab_hbm, itab_hbm, out_hbm,
              idx_u_v, idx_i_v, rows_u, rows_i, out_v,
              sem_i, sem_g):
    wid = lax.axis_index("s") * NUM_CORES + lax.axis_index("c")
    base = wid * B_PER_W

    cu = pltpu.async_copy(uidx_hbm.at[pl.ds(base, B_PER_W)], idx_u_v, sem_i)
    ci = pltpu.async_copy(iidx_hbm.at[pl.ds(base, B_PER_W)], idx_i_v, sem_i)
    cu.wait()
    ci.wait()

    def chunk_pass(c, carry):
        cbase = c * CHUNK
        cps = []
        for g16 in range(CHUNK // LANES):
            vu = idx_u_v[pl.ds(cbase + g16 * LANES, LANES)]
            vi = idx_i_v[pl.ds(cbase + g16 * LANES, LANES)]
            for j in range(LANES):
                r = g16 * LANES + j
                dst = (pl.ds(r, 1), slice(None))
                cps.append(pltpu.async_copy(
                    utab_hbm.at[pl.ds(vu[j], 1), :], rows_u.at[dst], sem_g))
                cps.append(pltpu.async_copy(
                    itab_hbm.at[pl.ds(vi[j], 1), :], rows_i.at[dst], sem_g))
        for cp in cps:
            cp.wait()

        for g in range(CGROUPS):
            row_ids = lax.iota(jnp.int32, LANES) + g * LANES
            acc = jnp.zeros((LANES,), jnp.float32)
            for d in range(EMBED_DIM):
                col = jnp.full((LANES,), d, jnp.int32)
                u = plsc.load_gather(rows_u, [row_ids, col])
                v = plsc.load_gather(rows_i, [row_ids, col])
                acc = acc + u * v
            out_v[pl.ds(cbase + g * LANES, LANES)] = acc
        return carry

    lax.fori_loop(0, NCHUNKS, chunk_pass, 0)

    pltpu.sync_copy(out_v, out_hbm.at[pl.ds(base, B_PER_W)])


def kernel(user_idx, item_idx, user_table, item_table):
    mesh = plsc.VectorSubcoreMesh(
        core_axis_name="c", subcore_axis_name="s",
        num_cores=NUM_CORES, num_subcores=NUM_SUBCORES)
    run = pl.kernel(
        _bpr_body,
        out_type=jax.ShapeDtypeStruct((BATCH,), jnp.float32),
        mesh=mesh,
        compiler_params=pltpu.CompilerParams(
            needs_layout_passes=False, use_tc_tiling_on_sc=True),
        scratch_types=[
            pltpu.VMEM((B_PER_W,), jnp.int32),
            pltpu.VMEM((B_PER_W,), jnp.int32),
            pltpu.VMEM((CHUNK, EMBED_DIM), jnp.float32),
            pltpu.VMEM((CHUNK, EMBED_DIM), jnp.float32),
            pltpu.VMEM((B_PER_W,), jnp.float32),
            pltpu.SemaphoreType.DMA,
            pltpu.SemaphoreType.DMA,
        ],
    )
    return run(user_idx, item_idx, user_table, item_table)
